# baseline (speedup 1.0000x reference)
import functools

import jax
import jax.numpy as jnp
from jax import lax
from jax.experimental import pallas as pl
from jax.experimental.pallas import tpu as pltpu
from jax.experimental.pallas import tpu_sc as plsc

NC = 2
NS = 16
NW = NC * NS
LANES = 16
FLUSH = 128
CB = 4096
PHALF = 2944
PEND = 2 * PHALF
NBLK = PEND // FLUSH
PADB = 2
PTOT = PEND + PADB * FLUSH
ROWS = 320
N_NODES = 10000
SENT = 1 << 30


def _matmul(features, W):
    n, d_in = features.shape
    d_out = W.shape[1]
    blk = 1000
    assert n % blk == 0

    def body(x_ref, w_ref, o_ref):
        o_ref[...] = jnp.dot(x_ref[...], w_ref[...],
                             preferred_element_type=jnp.float32
                             ).astype(jnp.bfloat16)

    return pl.pallas_call(
        body,
        grid=(n // blk,),
        in_specs=[
            pl.BlockSpec((blk, d_in), lambda i: (i, 0)),
            pl.BlockSpec((d_in, d_out), lambda i: (0, 0)),
        ],
        out_specs=pl.BlockSpec((blk, d_out), lambda i: (i, 0)),
        out_shape=jax.ShapeDtypeStruct((n, d_out), jnp.bfloat16),
    )(features, W)


def _make_scan(n_chunks):
    mesh = plsc.VectorSubcoreMesh(core_axis_name="c", subcore_axis_name="s")

    @functools.partial(
        pl.kernel,
        mesh=mesh,
        compiler_params=pltpu.CompilerParams(needs_layout_passes=False),
        out_type=jax.ShapeDtypeStruct((NW, 2, PTOT), jnp.int32),
        scratch_types=[
            pltpu.VMEM((CB,), jnp.int32),
            pltpu.VMEM((CB,), jnp.int32),
            pltpu.VMEM((CB,), jnp.int32),
            pltpu.VMEM((CB,), jnp.int32),
            pltpu.VMEM((PTOT + LANES,), jnp.int32),
            pltpu.VMEM((PTOT + LANES,), jnp.int32),
            pltpu.SemaphoreType.DMA,
            pltpu.SemaphoreType.DMA,
        ],
    )
    def scan(src_hbm, dst_hbm, pend_hbm, sch0, sch1, dch0, dch1,
             p_src, p_dl, ssem0, ssem1):
        cid = lax.axis_index("c")
        sid = lax.axis_index("s")
        wid = cid * NS + sid
        base = wid * ROWS
        my_rows = jnp.where(wid == NW - 1,
                            jnp.int32(N_NODES - (NW - 1) * ROWS),
                            jnp.int32(ROWS))

        zero16i = jnp.zeros((LANES,), jnp.int32)
        sent16 = jnp.broadcast_to(jnp.int32(SENT), (LANES,))
        base_v = jnp.broadcast_to(base, (LANES,))
        rows_u = jnp.broadcast_to(my_rows, (LANES,)).astype(jnp.uint32)
        limA = jnp.broadcast_to(jnp.int32(PHALF), (LANES,))
        limB = jnp.broadcast_to(jnp.int32(PEND), (LANES,))
        trash = jnp.broadcast_to(jnp.int32(PTOT), (LANES,))

        def zpend(i, c):
            p_src[pl.ds(i * LANES, LANES)] = zero16i
            p_dl[pl.ds(i * LANES, LANES)] = sent16
            return c

        lax.fori_loop(0, (PTOT + LANES) // LANES, zpend, 0)

        def scan_half(sch, dch, offA, offB):
            for g in range(CB // LANES):
                dv = dch[pl.ds(g * LANES, LANES)]
                sv = sch[pl.ds(g * LANES, LANES)]
                dl = dv - base_v
                mask = plsc.bitcast(dl, jnp.uint32) < rows_u
                mi = mask.astype(jnp.int32)
                csum = plsc.cumsum(mi)
                if g % 2 == 0:
                    pos = (jnp.broadcast_to(offA, (LANES,)) + csum) - mi
                    pos = jnp.where(pos < limA, pos, trash)
                    offA = offA + jnp.sum(mi)
                else:
                    pos = (jnp.broadcast_to(offB, (LANES,)) + csum) - mi
                    pos = jnp.where(pos < limB, pos, trash)
                    offB = offB + jnp.sum(mi)
                plsc.store_scatter(p_src, [pos], sv, mask=mask)
                plsc.store_scatter(p_dl, [pos], dl, mask=mask)
            return offA, offB

        pltpu.async_copy(src_hbm.at[pl.ds(0, CB)], sch0, ssem0)
        pltpu.async_copy(dst_hbm.at[pl.ds(0, CB)], dch0, ssem0)
        pltpu.async_copy(src_hbm.at[pl.ds(CB, CB)], sch1, ssem1)
        pltpu.async_copy(dst_hbm.at[pl.ds(CB, CB)], dch1, ssem1)

        def scan_pair(i, offs):
            offA, offB = offs
            cb = i * 2
            pltpu.make_async_copy(src_hbm.at[pl.ds(0, CB)], sch0, ssem0).wait()
            pltpu.make_async_copy(src_hbm.at[pl.ds(0, CB)], dch0, ssem0).wait()
            offA, offB = scan_half(sch0, dch0, offA, offB)

            @pl.when(cb + 2 < n_chunks)
            def _():
                pltpu.async_copy(src_hbm.at[pl.ds((cb + 2) * CB, CB)],
                                 sch0, ssem0)
                pltpu.async_copy(dst_hbm.at[pl.ds((cb + 2) * CB, CB)],
                                 dch0, ssem0)

            pltpu.make_async_copy(src_hbm.at[pl.ds(0, CB)], sch1, ssem1).wait()
            pltpu.make_async_copy(src_hbm.at[pl.ds(0, CB)], dch1, ssem1).wait()
            offA, offB = scan_half(sch1, dch1, offA, offB)

            @pl.when(cb + 3 < n_chunks)
            def _():
                pltpu.async_copy(src_hbm.at[pl.ds((cb + 3) * CB, CB)],
                                 sch1, ssem1)
                pltpu.async_copy(dst_hbm.at[pl.ds((cb + 3) * CB, CB)],
                                 dch1, ssem1)

            return offA, offB

        lax.fori_loop(0, n_chunks // 2, scan_pair,
                      (jnp.int32(0), jnp.int32(PHALF)))

        pltpu.sync_copy(p_src.at[pl.ds(0, PTOT)],
                        pend_hbm.at[wid, 0, pl.ds(0, PTOT)])
        pltpu.sync_copy(p_dl.at[pl.ds(0, PTOT)],
                        pend_hbm.at[wid, 1, pl.ds(0, PTOT)])

    return scan


def _make_accumulate(n_nodes, d):
    last_rows = n_nodes - (NW - 1) * ROWS
    assert 0 < last_rows <= ROWS and last_rows % 8 == 0

    mesh = plsc.VectorSubcoreMesh(core_axis_name="c", subcore_axis_name="s")

    @functools.partial(
        pl.kernel,
        mesh=mesh,
        compiler_params=pltpu.CompilerParams(needs_layout_passes=False),
        out_type=jax.ShapeDtypeStruct((n_nodes * d,), jnp.float32),
        scratch_types=[
            pltpu.VMEM((PTOT,), jnp.int32),
            pltpu.VMEM((PTOT,), jnp.int32),
            pltpu.VMEM((FLUSH, d // 2), jnp.int32),
            pltpu.VMEM((FLUSH, d // 2), jnp.int32),
            pltpu.VMEM(((ROWS + 1) * d,), jnp.float32),
            pltpu.SemaphoreType.DMA,
            pltpu.SemaphoreType.DMA,
        ],
    )
    def accumulate(support_hbm, psrc_hbm, pdl_hbm, out_hbm,
                   bidx, bdl, buf0, buf1, acc_v, sem0, sem1):
        cid = lax.axis_index("c")
        sid = lax.axis_index("s")
        wid = cid * NS + sid
        base = wid * ROWS

        zero16f = jnp.zeros((LANES,), jnp.float32)

        def zacc(i, c):
            acc_v[pl.ds(i * LANES, LANES)] = zero16f
            return c

        lax.fori_loop(0, (ROWS + 1) * d // LANES, zacc, 0)

        HL = 2 * LANES

        def acc_block(buf, b):
            def acc_eg(eg, c):
                dv16 = jnp.minimum(
                    bdl[pl.ds(b * FLUSH + eg * LANES, LANES)],
                    jnp.int32(ROWS))
                for e16 in range(LANES):
                    e = eg * LANES + e16
                    rb = pl.multiple_of(dv16[e16] * d, d)
                    for k in range(d // HL):
                        vi = buf[e, pl.ds(k * LANES, LANES)]
                        v32 = plsc.bitcast(vi, jnp.bfloat16)
                        lo, hi = plsc.unpack(
                            v32, format=plsc.PackFormat.INTERLEAVED,
                            preferred_element_type=jnp.float32)
                        plsc.addupdate(
                            acc_v.at[pl.ds(rb + k * HL, LANES)], lo)
                        plsc.addupdate(
                            acc_v.at[pl.ds(rb + k * HL + LANES, LANES)], hi)
                return c

            lax.fori_loop(0, FLUSH // LANES, acc_eg, 0)

        def gather(b, buf, sem):
            pltpu.async_copy(
                support_hbm.at[bidx.at[pl.ds(b * FLUSH, FLUSH)]], buf, sem)

        pltpu.sync_copy(psrc_hbm.at[wid], bidx)
        pltpu.sync_copy(pdl_hbm.at[wid], bdl)
        gather(0, buf0, sem0)
        gather(1, buf1, sem1)

        def pair(i, c):
            b = i * 2
            pltpu.make_async_copy(support_hbm.at[bidx.at[pl.ds(0, FLUSH)]],
                                  buf0, sem0).wait()
            acc_block(buf0, b)
            gather(b + 2, buf0, sem0)
            pltpu.make_async_copy(support_hbm.at[bidx.at[pl.ds(0, FLUSH)]],
                                  buf1, sem1).wait()
            acc_block(buf1, b + 1)
            gather(b + 3, buf1, sem1)
            return c

        lax.fori_loop(0, NBLK // 2, pair, 0)

        pltpu.make_async_copy(support_hbm.at[bidx.at[pl.ds(0, FLUSH)]],
                              buf0, sem0).wait()
        pltpu.make_async_copy(support_hbm.at[bidx.at[pl.ds(0, FLUSH)]],
                              buf1, sem1).wait()

        @pl.when(wid < NW - 1)
        def _wb():
            pltpu.sync_copy(acc_v.at[pl.ds(0, ROWS * d)],
                            out_hbm.at[pl.ds(base * d, ROWS * d)])

        @pl.when(wid == NW - 1)
        def _wb_last():
            pltpu.sync_copy(acc_v.at[pl.ds(0, last_rows * d)],
                            out_hbm.at[pl.ds(base * d, last_rows * d)])

    return accumulate


def kernel(features, edge_index, W):
    n_nodes, d_in = features.shape
    d = W.shape[1]
    assert n_nodes == N_NODES
    g32 = jnp.arange(d) // 32
    p32 = jnp.arange(d) % 32
    perm = g32 * 32 + jnp.where(p32 % 2 == 0, p32 // 2, 16 + p32 // 2)
    support = _matmul(features, W[:, perm])
    support = jax.lax.bitcast_convert_type(
        support.reshape(n_nodes, d // 2, 2), jnp.int32)

    dst = edge_index[0].astype(jnp.int32)
    src = edge_index[1].astype(jnp.int32)
    n_edges = dst.shape[0]

    n_chunks = -(-n_edges // CB)
    n_chunks = n_chunks + (n_chunks % 2)
    pad = n_chunks * CB - n_edges
    dst_p = jnp.concatenate([dst, jnp.full((pad,), SENT, jnp.int32)])
    src_p = jnp.concatenate([src, jnp.zeros((pad,), jnp.int32)])

    pend = _make_scan(n_chunks)(src_p, dst_p)
    psrc = pend[:, 0]
    pdl = pend[:, 1]
    out_flat = _make_accumulate(n_nodes, d)(support, psrc, pdl)
    return out_flat.reshape(n_nodes, d)


# speedup vs baseline: 1.1219x; 1.1219x over previous
import functools

import jax
import jax.numpy as jnp
from jax import lax
from jax.experimental import pallas as pl
from jax.experimental.pallas import tpu as pltpu
from jax.experimental.pallas import tpu_sc as plsc

NC = 2
NS = 16
NW = NC * NS
LANES = 16
FLUSH = 64
CB = 2048
PHALF = 2944
PEND = 2 * PHALF
NBLK = PEND // FLUSH
PADB = 2
PTOT = PEND + PADB * FLUSH
ROWS = 320
N_NODES = 10000
SENT = 1 << 30


def _matmul(features, W):
    n, d_in = features.shape
    d_out = W.shape[1]
    blk = 1000
    assert n % blk == 0

    def body(x_ref, w_ref, o_ref):
        o_ref[...] = jnp.dot(x_ref[...], w_ref[...],
                             preferred_element_type=jnp.float32
                             ).astype(jnp.bfloat16)

    return pl.pallas_call(
        body,
        grid=(n // blk,),
        in_specs=[
            pl.BlockSpec((blk, d_in), lambda i: (i, 0)),
            pl.BlockSpec((d_in, d_out), lambda i: (0, 0)),
        ],
        out_specs=pl.BlockSpec((blk, d_out), lambda i: (i, 0)),
        out_shape=jax.ShapeDtypeStruct((n, d_out), jnp.bfloat16),
    )(features, W)


def _make_scan(n_chunks):
    mesh = plsc.VectorSubcoreMesh(core_axis_name="c", subcore_axis_name="s")

    @functools.partial(
        pl.kernel,
        mesh=mesh,
        compiler_params=pltpu.CompilerParams(needs_layout_passes=False),
        out_type=jax.ShapeDtypeStruct((N_NODES * 256,), jnp.float32),
        scratch_types=[
            pltpu.VMEM((CB,), jnp.int32),
            pltpu.VMEM((CB,), jnp.int32),
            pltpu.VMEM((CB,), jnp.int32),
            pltpu.VMEM((CB,), jnp.int32),
            pltpu.VMEM((PTOT + LANES,), jnp.int32),
            pltpu.VMEM((PTOT + LANES,), jnp.int32),
            pltpu.VMEM((FLUSH, 128), jnp.int32),
            pltpu.VMEM((FLUSH, 128), jnp.int32),
            pltpu.VMEM(((ROWS + 1) * 256,), jnp.float32),
            pltpu.SemaphoreType.DMA,
            pltpu.SemaphoreType.DMA,
            pltpu.SemaphoreType.DMA,
            pltpu.SemaphoreType.DMA,
        ],
    )
    def scan(support_hbm, src_hbm, dst_hbm, out_hbm, sch0, sch1, dch0, dch1,
             p_src, p_dl, buf0, buf1, acc_v, ssem0, ssem1, sem0, sem1):
        d = 256
        HL = 2 * LANES
        last_rows = N_NODES - (NW - 1) * ROWS
        cid = lax.axis_index("c")
        sid = lax.axis_index("s")
        wid = cid * NS + sid
        base = wid * ROWS
        my_rows = jnp.where(wid == NW - 1,
                            jnp.int32(N_NODES - (NW - 1) * ROWS),
                            jnp.int32(ROWS))

        zero16i = jnp.zeros((LANES,), jnp.int32)
        sent16 = jnp.broadcast_to(jnp.int32(SENT), (LANES,))
        base_v = jnp.broadcast_to(base, (LANES,))
        rows_u = jnp.broadcast_to(my_rows, (LANES,)).astype(jnp.uint32)
        limA = jnp.broadcast_to(jnp.int32(PHALF), (LANES,))
        limB = jnp.broadcast_to(jnp.int32(PEND), (LANES,))
        trash = jnp.broadcast_to(jnp.int32(PTOT), (LANES,))

        def zpend(i, c):
            p_src[pl.ds(i * LANES, LANES)] = zero16i
            p_dl[pl.ds(i * LANES, LANES)] = sent16
            return c

        lax.fori_loop(0, (PTOT + LANES) // LANES, zpend, 0)

        zero16f = jnp.zeros((LANES,), jnp.float32)

        def zacc(i, c):
            acc_v[pl.ds(i * LANES, LANES)] = zero16f
            return c

        lax.fori_loop(0, (ROWS + 1) * d // LANES, zacc, 0)

        def scan_half(sch, dch, offA, offB):
            for g in range(CB // LANES):
                dv = dch[pl.ds(g * LANES, LANES)]
                sv = sch[pl.ds(g * LANES, LANES)]
                dl = dv - base_v
                mask = plsc.bitcast(dl, jnp.uint32) < rows_u
                mi = mask.astype(jnp.int32)
                csum = plsc.cumsum(mi)
                if g % 2 == 0:
                    pos = (jnp.broadcast_to(offA, (LANES,)) + csum) - mi
                    pos = jnp.where(pos < limA, pos, trash)
                    offA = offA + jnp.sum(mi)
                else:
                    pos = (jnp.broadcast_to(offB, (LANES,)) + csum) - mi
                    pos = jnp.where(pos < limB, pos, trash)
                    offB = offB + jnp.sum(mi)
                plsc.store_scatter(p_src, [pos], sv, mask=mask)
                plsc.store_scatter(p_dl, [pos], dl, mask=mask)
            return offA, offB

        pltpu.async_copy(src_hbm.at[pl.ds(0, CB)], sch0, ssem0)
        pltpu.async_copy(dst_hbm.at[pl.ds(0, CB)], dch0, ssem0)
        pltpu.async_copy(src_hbm.at[pl.ds(CB, CB)], sch1, ssem1)
        pltpu.async_copy(dst_hbm.at[pl.ds(CB, CB)], dch1, ssem1)

        def scan_pair(i, offs):
            offA, offB = offs
            cb = i * 2
            pltpu.make_async_copy(src_hbm.at[pl.ds(0, CB)], sch0, ssem0).wait()
            pltpu.make_async_copy(src_hbm.at[pl.ds(0, CB)], dch0, ssem0).wait()
            offA, offB = scan_half(sch0, dch0, offA, offB)

            @pl.when(cb + 2 < n_chunks)
            def _():
                pltpu.async_copy(src_hbm.at[pl.ds((cb + 2) * CB, CB)],
                                 sch0, ssem0)
                pltpu.async_copy(dst_hbm.at[pl.ds((cb + 2) * CB, CB)],
                                 dch0, ssem0)

            pltpu.make_async_copy(src_hbm.at[pl.ds(0, CB)], sch1, ssem1).wait()
            pltpu.make_async_copy(src_hbm.at[pl.ds(0, CB)], dch1, ssem1).wait()
            offA, offB = scan_half(sch1, dch1, offA, offB)

            @pl.when(cb + 3 < n_chunks)
            def _():
                pltpu.async_copy(src_hbm.at[pl.ds((cb + 3) * CB, CB)],
                                 sch1, ssem1)
                pltpu.async_copy(dst_hbm.at[pl.ds((cb + 3) * CB, CB)],
                                 dch1, ssem1)

            return offA, offB

        lax.fori_loop(0, n_chunks // 2, scan_pair,
                      (jnp.int32(0), jnp.int32(PHALF)))

        def acc_block(buf, b):
            def acc_eg(eg, c):
                dv16 = jnp.minimum(
                    p_dl[pl.ds(b * FLUSH + eg * LANES, LANES)],
                    jnp.int32(ROWS))
                for e0 in range(0, LANES, 2):
                    ea, eb = eg * LANES + e0, eg * LANES + e0 + 1
                    rba = pl.multiple_of(dv16[e0] * d, d)
                    rbb = pl.multiple_of(dv16[e0 + 1] * d, d)
                    for k in range(d // HL):
                        via = buf[ea, pl.ds(k * LANES, LANES)]
                        vib = buf[eb, pl.ds(k * LANES, LANES)]
                        va = plsc.bitcast(via, jnp.bfloat16)
                        vb = plsc.bitcast(vib, jnp.bfloat16)
                        loa, hia = plsc.unpack(
                            va, format=plsc.PackFormat.INTERLEAVED,
                            preferred_element_type=jnp.float32)
                        lob, hib = plsc.unpack(
                            vb, format=plsc.PackFormat.INTERLEAVED,
                            preferred_element_type=jnp.float32)
                        plsc.addupdate(
                            acc_v.at[pl.ds(rba + k * HL, LANES)], loa)
                        plsc.addupdate(
                            acc_v.at[pl.ds(rba + k * HL + LANES, LANES)], hia)
                        plsc.addupdate(
                            acc_v.at[pl.ds(rbb + k * HL, LANES)], lob)
                        plsc.addupdate(
                            acc_v.at[pl.ds(rbb + k * HL + LANES, LANES)], hib)
                return c

            lax.fori_loop(0, FLUSH // LANES, acc_eg, 0)

        def gather(b, buf, sem):
            pltpu.async_copy(
                support_hbm.at[p_src.at[pl.ds(b * FLUSH, FLUSH)]], buf, sem)

        gather(0, buf0, sem0)
        gather(1, buf1, sem1)

        def pair(i, c):
            b = i * 2
            pltpu.make_async_copy(support_hbm.at[p_src.at[pl.ds(0, FLUSH)]],
                                  buf0, sem0).wait()
            acc_block(buf0, b)
            gather(b + 2, buf0, sem0)
            pltpu.make_async_copy(support_hbm.at[p_src.at[pl.ds(0, FLUSH)]],
                                  buf1, sem1).wait()
            acc_block(buf1, b + 1)
            gather(b + 3, buf1, sem1)
            return c

        lax.fori_loop(0, NBLK // 2, pair, 0)

        pltpu.make_async_copy(support_hbm.at[p_src.at[pl.ds(0, FLUSH)]],
                              buf0, sem0).wait()
        pltpu.make_async_copy(support_hbm.at[p_src.at[pl.ds(0, FLUSH)]],
                              buf1, sem1).wait()

        @pl.when(wid < NW - 1)
        def _wb():
            pltpu.sync_copy(acc_v.at[pl.ds(0, ROWS * d)],
                            out_hbm.at[pl.ds(base * d, ROWS * d)])

        @pl.when(wid == NW - 1)
        def _wb_last():
            pltpu.sync_copy(acc_v.at[pl.ds(0, last_rows * d)],
                            out_hbm.at[pl.ds(base * d, last_rows * d)])

    return scan


def kernel(features, edge_index, W):
    n_nodes, d_in = features.shape
    d = W.shape[1]
    assert n_nodes == N_NODES
    g32 = jnp.arange(d) // 32
    p32 = jnp.arange(d) % 32
    perm = g32 * 32 + jnp.where(p32 % 2 == 0, p32 // 2, 16 + p32 // 2)
    support = _matmul(features, W[:, perm])
    support = jax.lax.bitcast_convert_type(
        support.reshape(n_nodes, d // 2, 2), jnp.int32)

    dst = edge_index[0].astype(jnp.int32)
    src = edge_index[1].astype(jnp.int32)
    n_edges = dst.shape[0]

    n_chunks = -(-n_edges // CB)
    n_chunks = n_chunks + (n_chunks % 2)
    pad = n_chunks * CB - n_edges
    dst_p = jnp.concatenate([dst, jnp.full((pad,), SENT, jnp.int32)])
    src_p = jnp.concatenate([src, jnp.zeros((pad,), jnp.int32)])

    out_flat = _make_scan(n_chunks)(support, src_p, dst_p)
    return out_flat.reshape(n_nodes, d)
